# baseline (device time: 43089 ns/iter reference)
import jax
import jax.numpy as jnp
from jax import lax
from jax.experimental import pallas as pl
from jax.experimental.pallas import tpu as pltpu

N_DEV = 4
_BF = jnp.bfloat16


def kernel(x, Win0, Wout0, Win1, Wout1, Win2, Wout2):
    B, d_sh = x.shape
    H = Win0.shape[1]
    Bq = B // N_DEV

    def body(x_hbm, win0_hbm, wout0_hbm, win1_hbm, wout1_hbm, win2_hbm,
             wout2_hbm, out_ref, xv, winv, woutv, p_send, rs_buf,
             hown_buf, hg_buf, send_sems, recv_sems, in_sems):
        me = lax.axis_index("i")

        in_copies = []
        win_hbm = [win0_hbm, win1_hbm, win2_hbm]
        wout_hbm = [wout0_hbm, wout1_hbm, wout2_hbm]
        cp_x = pltpu.make_async_copy(x_hbm, xv, in_sems.at[0])
        cp_x.start()
        cp_win = []
        cp_wout = []
        for L in range(3):
            c = pltpu.make_async_copy(win_hbm[L], winv.at[L], in_sems.at[1 + L])
            c.start()
            cp_win.append(c)
            c = pltpu.make_async_copy(wout_hbm[L], woutv.at[L], in_sems.at[4 + L])
            c.start()
            cp_wout.append(c)

        bsem = pltpu.get_barrier_semaphore()
        for d in range(1, N_DEV):
            pl.semaphore_signal(
                bsem, inc=1,
                device_id=((me + d) % N_DEV,),
                device_id_type=pl.DeviceIdType.MESH,
            )
        pl.semaphore_wait(bsem, N_DEV - 1)

        sends = []

        def rs_send(L, dd, blk_bf):
            tgt = (me + dd) % N_DEV
            p_send[L, dd] = blk_bf
            rdma = pltpu.make_async_remote_copy(
                src_ref=p_send.at[L, dd],
                dst_ref=rs_buf.at[L, N_DEV - dd],
                send_sem=send_sems.at[2 * L, dd],
                recv_sem=recv_sems.at[2 * L, N_DEV - dd],
                device_id=(tgt,),
                device_id_type=pl.DeviceIdType.MESH,
            )
            rdma.start()
            sends.append(rdma)

        def wait_recv(phase, dd, buf_ref):
            recv = pltpu.make_async_remote_copy(
                src_ref=buf_ref, dst_ref=buf_ref,
                send_sem=send_sems.at[phase, dd],
                recv_sem=recv_sems.at[phase, dd],
                device_id=(me,),
                device_id_type=pl.DeviceIdType.MESH,
            )
            recv.wait_recv()

        cp_x.wait()
        cp_win[0].wait()
        w_in0 = winv[0].astype(_BF)
        for dd in (2, 1, 3):
            s = (me + dd) % N_DEV
            blk = jnp.dot(
                xv[pl.ds(s * Bq, Bq), :].astype(_BF), w_in0,
                preferred_element_type=jnp.float32,
            ).astype(_BF)
            rs_send(0, dd, blk)
        acc = jnp.dot(
            xv[pl.ds(me * Bq, Bq), :].astype(_BF), w_in0,
            preferred_element_type=jnp.float32,
        )

        for L in range(3):
            p_ag = 2 * L + 1

            for dd in range(1, N_DEV):
                wait_recv(2 * L, dd, rs_buf.at[L, dd])
                acc = acc + rs_buf[L, dd].astype(jnp.float32)
            hown_buf[L] = jnp.maximum(acc, 0.0).astype(_BF)

            for dd in (2, 1, 3):
                tgt = (me + dd) % N_DEV
                rdma = pltpu.make_async_remote_copy(
                    src_ref=hown_buf.at[L],
                    dst_ref=hg_buf.at[L, N_DEV - dd],
                    send_sem=send_sems.at[p_ag, dd],
                    recv_sem=recv_sems.at[p_ag, N_DEV - dd],
                    device_id=(tgt,),
                    device_id_type=pl.DeviceIdType.MESH,
                )
                rdma.start()
                sends.append(rdma)

            cp_wout[L].wait()
            w_out = woutv[L].astype(_BF)
            if L < 2:
                cp_win[L + 1].wait()
                w_in_next = winv[L + 1].astype(_BF)

            xnext_own = jnp.dot(
                hown_buf[L], w_out, preferred_element_type=jnp.float32
            )
            if L < 2:
                acc_next = jnp.dot(
                    xnext_own.astype(_BF), w_in_next,
                    preferred_element_type=jnp.float32,
                )
            else:
                out_ref[pl.ds(me * Bq, Bq), :] = xnext_own

            for dd in range(1, N_DEV):
                wait_recv(p_ag, dd, hg_buf.at[L, dd])
                s = (me + dd) % N_DEV
                xnext_s = jnp.dot(
                    hg_buf[L, dd], w_out, preferred_element_type=jnp.float32
                )
                if L < 2:
                    blk = jnp.dot(
                        xnext_s.astype(_BF), w_in_next,
                        preferred_element_type=jnp.float32,
                    ).astype(_BF)
                    rs_send(L + 1, dd, blk)
                else:
                    out_ref[pl.ds(s * Bq, Bq), :] = xnext_s

            if L < 2:
                acc = acc_next

        for rdma in sends:
            rdma.wait_send()

    return pl.pallas_call(
        body,
        out_shape=jax.ShapeDtypeStruct((B, d_sh), jnp.float32),
        in_specs=[pl.BlockSpec(memory_space=pltpu.MemorySpace.HBM)] * 7,
        out_specs=pl.BlockSpec(memory_space=pltpu.VMEM),
        scratch_shapes=[
            pltpu.VMEM((B, d_sh), jnp.float32),
            pltpu.VMEM((3, d_sh, H), jnp.float32),
            pltpu.VMEM((3, H, d_sh), jnp.float32),
            pltpu.VMEM((3, N_DEV, Bq, H), _BF),
            pltpu.VMEM((3, N_DEV, Bq, H), _BF),
            pltpu.VMEM((3, Bq, H), _BF),
            pltpu.VMEM((3, N_DEV, Bq, H), _BF),
            pltpu.SemaphoreType.DMA((6, N_DEV)),
            pltpu.SemaphoreType.DMA((6, N_DEV)),
            pltpu.SemaphoreType.DMA((7,)),
        ],
        compiler_params=pltpu.CompilerParams(collective_id=0),
    )(x, Win0, Wout0, Win1, Wout1, Win2, Wout2)


# device time: 35714 ns/iter; 1.2065x vs baseline; 1.2065x over previous
import jax
import jax.numpy as jnp
from jax import lax
from jax.experimental import pallas as pl
from jax.experimental.pallas import tpu as pltpu

N_DEV = 4
N_SUB = 2
_BF = jnp.bfloat16


def kernel(x, Win0, Wout0, Win1, Wout1, Win2, Wout2):
    B, d_sh = x.shape
    H = Win0.shape[1]
    Bq = B // N_DEV
    Bs = Bq // N_SUB

    def body(x_ref, win0_ref, wout0_ref, win1_ref, wout1_ref, win2_ref,
             wout2_ref, out_ref, p_send, rs_buf, hown_buf, hg_buf,
             send_sems, recv_sems):
        me = lax.axis_index("i")

        bsem = pltpu.get_barrier_semaphore()
        for d in range(1, N_DEV):
            pl.semaphore_signal(
                bsem, inc=1,
                device_id=((me + d) % N_DEV,),
                device_id_type=pl.DeviceIdType.MESH,
            )
        pl.semaphore_wait(bsem, N_DEV - 1)

        win_refs = [win0_ref, win1_ref, win2_ref]
        wout_refs = [wout0_ref, wout1_ref, wout2_ref]
        sends = []

        def rs_send(L, sub, dd, blk_bf):
            tgt = (me + dd) % N_DEV
            p_send[L, sub, dd] = blk_bf
            rdma = pltpu.make_async_remote_copy(
                src_ref=p_send.at[L, sub, dd],
                dst_ref=rs_buf.at[L, sub, N_DEV - dd],
                send_sem=send_sems.at[2 * L, sub, dd],
                recv_sem=recv_sems.at[2 * L, sub, N_DEV - dd],
                device_id=(tgt,),
                device_id_type=pl.DeviceIdType.MESH,
            )
            rdma.start()
            sends.append(rdma)

        def wait_recv(phase, sub, dd, buf_ref):
            recv = pltpu.make_async_remote_copy(
                src_ref=buf_ref, dst_ref=buf_ref,
                send_sem=send_sems.at[phase, sub, dd],
                recv_sem=recv_sems.at[phase, sub, dd],
                device_id=(me,),
                device_id_type=pl.DeviceIdType.MESH,
            )
            recv.wait_recv()

        w_in0 = win_refs[0][...].astype(_BF)
        for sub in range(N_SUB):
            for dd in (2, 1, 3):
                s = (me + dd) % N_DEV
                blk = jnp.dot(
                    x_ref[pl.ds(s * Bq + sub * Bs, Bs), :].astype(_BF),
                    w_in0, preferred_element_type=jnp.float32,
                ).astype(_BF)
                rs_send(0, sub, dd, blk)
        acc = [
            jnp.dot(
                x_ref[pl.ds(me * Bq + sub * Bs, Bs), :].astype(_BF),
                w_in0, preferred_element_type=jnp.float32,
            )
            for sub in range(N_SUB)
        ]

        for L in range(3):
            p_ag = 2 * L + 1
            w_out = wout_refs[L][...].astype(_BF)
            w_in_next = win_refs[L + 1][...].astype(_BF) if L < 2 else None
            acc_next = [None] * N_SUB

            for sub in range(N_SUB):
                a = acc[sub]
                for dd in range(1, N_DEV):
                    wait_recv(2 * L, sub, dd, rs_buf.at[L, sub, dd])
                    a = a + rs_buf[L, sub, dd].astype(jnp.float32)
                hown_buf[L, sub] = jnp.maximum(a, 0.0).astype(_BF)

                for dd in (2, 1, 3):
                    tgt = (me + dd) % N_DEV
                    rdma = pltpu.make_async_remote_copy(
                        src_ref=hown_buf.at[L, sub],
                        dst_ref=hg_buf.at[L, sub, N_DEV - dd],
                        send_sem=send_sems.at[p_ag, sub, dd],
                        recv_sem=recv_sems.at[p_ag, sub, N_DEV - dd],
                        device_id=(tgt,),
                        device_id_type=pl.DeviceIdType.MESH,
                    )
                    rdma.start()
                    sends.append(rdma)

                xnext_own = jnp.dot(
                    hown_buf[L, sub], w_out,
                    preferred_element_type=jnp.float32,
                )
                if L < 2:
                    acc_next[sub] = jnp.dot(
                        xnext_own.astype(_BF), w_in_next,
                        preferred_element_type=jnp.float32,
                    )
                else:
                    out_ref[pl.ds(me * Bq + sub * Bs, Bs), :] = xnext_own

            for sub in range(N_SUB):
                for dd in range(1, N_DEV):
                    wait_recv(p_ag, sub, dd, hg_buf.at[L, sub, dd])
                    s = (me + dd) % N_DEV
                    xnext_s = jnp.dot(
                        hg_buf[L, sub, dd], w_out,
                        preferred_element_type=jnp.float32,
                    )
                    if L < 2:
                        blk = jnp.dot(
                            xnext_s.astype(_BF), w_in_next,
                            preferred_element_type=jnp.float32,
                        ).astype(_BF)
                        rs_send(L + 1, sub, dd, blk)
                    else:
                        out_ref[pl.ds(s * Bq + sub * Bs, Bs), :] = xnext_s

            if L < 2:
                acc = acc_next

        for rdma in sends:
            rdma.wait_send()

    return pl.pallas_call(
        body,
        out_shape=jax.ShapeDtypeStruct((B, d_sh), jnp.float32),
        in_specs=[pl.BlockSpec(memory_space=pltpu.VMEM)] * 7,
        out_specs=pl.BlockSpec(memory_space=pltpu.VMEM),
        scratch_shapes=[
            pltpu.VMEM((3, N_SUB, N_DEV, Bs, H), _BF),
            pltpu.VMEM((3, N_SUB, N_DEV, Bs, H), _BF),
            pltpu.VMEM((3, N_SUB, Bs, H), _BF),
            pltpu.VMEM((3, N_SUB, N_DEV, Bs, H), _BF),
            pltpu.SemaphoreType.DMA((6, N_SUB, N_DEV)),
            pltpu.SemaphoreType.DMA((6, N_SUB, N_DEV)),
        ],
        compiler_params=pltpu.CompilerParams(collective_id=0),
    )(x, Win0, Wout0, Win1, Wout1, Win2, Wout2)


# device time: 35396 ns/iter; 1.2173x vs baseline; 1.0090x over previous
import jax
import jax.numpy as jnp
from jax import lax
from jax.experimental import pallas as pl
from jax.experimental.pallas import tpu as pltpu

N_DEV = 4
N_SUB = 4
_BF = jnp.bfloat16


def kernel(x, Win0, Wout0, Win1, Wout1, Win2, Wout2):
    B, d_sh = x.shape
    H = Win0.shape[1]
    Bq = B // N_DEV
    Bs = Bq // N_SUB

    def body(x_ref, win0_ref, wout0_ref, win1_ref, wout1_ref, win2_ref,
             wout2_ref, out_ref, p_send, rs_buf, hown_buf, hg_buf,
             send_sems, recv_sems):
        me = lax.axis_index("i")

        bsem = pltpu.get_barrier_semaphore()
        for d in range(1, N_DEV):
            pl.semaphore_signal(
                bsem, inc=1,
                device_id=((me + d) % N_DEV,),
                device_id_type=pl.DeviceIdType.MESH,
            )
        pl.semaphore_wait(bsem, N_DEV - 1)

        win_refs = [win0_ref, win1_ref, win2_ref]
        wout_refs = [wout0_ref, wout1_ref, wout2_ref]
        sends = []

        def rs_send(L, sub, dd, blk_bf):
            tgt = (me + dd) % N_DEV
            p_send[L, sub, dd] = blk_bf
            rdma = pltpu.make_async_remote_copy(
                src_ref=p_send.at[L, sub, dd],
                dst_ref=rs_buf.at[L, sub, N_DEV - dd],
                send_sem=send_sems.at[2 * L, sub, dd],
                recv_sem=recv_sems.at[2 * L, sub, N_DEV - dd],
                device_id=(tgt,),
                device_id_type=pl.DeviceIdType.MESH,
            )
            rdma.start()
            sends.append(rdma)

        def wait_recv(phase, sub, dd, buf_ref):
            recv = pltpu.make_async_remote_copy(
                src_ref=buf_ref, dst_ref=buf_ref,
                send_sem=send_sems.at[phase, sub, dd],
                recv_sem=recv_sems.at[phase, sub, dd],
                device_id=(me,),
                device_id_type=pl.DeviceIdType.MESH,
            )
            recv.wait_recv()

        w_in0 = win_refs[0][...].astype(_BF)
        for sub in range(N_SUB):
            for dd in (2, 1, 3):
                s = (me + dd) % N_DEV
                blk = jnp.dot(
                    x_ref[pl.ds(s * Bq + sub * Bs, Bs), :].astype(_BF),
                    w_in0, preferred_element_type=jnp.float32,
                ).astype(_BF)
                rs_send(0, sub, dd, blk)
        acc = [
            jnp.dot(
                x_ref[pl.ds(me * Bq + sub * Bs, Bs), :].astype(_BF),
                w_in0, preferred_element_type=jnp.float32,
            )
            for sub in range(N_SUB)
        ]

        for L in range(3):
            p_ag = 2 * L + 1
            w_out = wout_refs[L][...].astype(_BF)
            w_in_next = win_refs[L + 1][...].astype(_BF) if L < 2 else None
            acc_next = [None] * N_SUB

            for sub in range(N_SUB):
                a = acc[sub]
                for dd in range(1, N_DEV):
                    wait_recv(2 * L, sub, dd, rs_buf.at[L, sub, dd])
                    a = a + rs_buf[L, sub, dd].astype(jnp.float32)
                hown_buf[L, sub] = jnp.maximum(a, 0.0).astype(_BF)

                for dd in (2, 1, 3):
                    tgt = (me + dd) % N_DEV
                    rdma = pltpu.make_async_remote_copy(
                        src_ref=hown_buf.at[L, sub],
                        dst_ref=hg_buf.at[L, sub, N_DEV - dd],
                        send_sem=send_sems.at[p_ag, sub, dd],
                        recv_sem=recv_sems.at[p_ag, sub, N_DEV - dd],
                        device_id=(tgt,),
                        device_id_type=pl.DeviceIdType.MESH,
                    )
                    rdma.start()
                    sends.append(rdma)

                xnext_own = jnp.dot(
                    hown_buf[L, sub], w_out,
                    preferred_element_type=jnp.float32,
                )
                if L < 2:
                    acc_next[sub] = jnp.dot(
                        xnext_own.astype(_BF), w_in_next,
                        preferred_element_type=jnp.float32,
                    )
                else:
                    out_ref[pl.ds(me * Bq + sub * Bs, Bs), :] = xnext_own

            for sub in range(N_SUB):
                for dd in range(1, N_DEV):
                    wait_recv(p_ag, sub, dd, hg_buf.at[L, sub, dd])
                    s = (me + dd) % N_DEV
                    xnext_s = jnp.dot(
                        hg_buf[L, sub, dd], w_out,
                        preferred_element_type=jnp.float32,
                    )
                    if L < 2:
                        blk = jnp.dot(
                            xnext_s.astype(_BF), w_in_next,
                            preferred_element_type=jnp.float32,
                        ).astype(_BF)
                        rs_send(L + 1, sub, dd, blk)
                    else:
                        out_ref[pl.ds(s * Bq + sub * Bs, Bs), :] = xnext_s

            if L < 2:
                acc = acc_next

        for rdma in sends:
            rdma.wait_send()

    return pl.pallas_call(
        body,
        out_shape=jax.ShapeDtypeStruct((B, d_sh), jnp.float32),
        in_specs=[pl.BlockSpec(memory_space=pltpu.VMEM)] * 7,
        out_specs=pl.BlockSpec(memory_space=pltpu.VMEM),
        scratch_shapes=[
            pltpu.VMEM((3, N_SUB, N_DEV, Bs, H), _BF),
            pltpu.VMEM((3, N_SUB, N_DEV, Bs, H), _BF),
            pltpu.VMEM((3, N_SUB, Bs, H), _BF),
            pltpu.VMEM((3, N_SUB, N_DEV, Bs, H), _BF),
            pltpu.SemaphoreType.DMA((6, N_SUB, N_DEV)),
            pltpu.SemaphoreType.DMA((6, N_SUB, N_DEV)),
        ],
        compiler_params=pltpu.CompilerParams(collective_id=0),
    )(x, Win0, Wout0, Win1, Wout1, Win2, Wout2)


# device time: 34513 ns/iter; 1.2485x vs baseline; 1.0256x over previous
import jax
import jax.numpy as jnp
from jax import lax
from jax.experimental import pallas as pl
from jax.experimental.pallas import tpu as pltpu

N_DEV = 4
N_SUB = 4
_BF = jnp.bfloat16


def kernel(x, Win0, Wout0, Win1, Wout1, Win2, Wout2):
    B, d_sh = x.shape
    H = Win0.shape[1]
    Bq = B // N_DEV
    Bs = Bq // N_SUB

    def body(x_ref, win0_ref, wout0_ref, win1_ref, wout1_ref, win2_ref,
             wout2_ref, out_ref, p_send, rs_buf, hown_buf, hg_buf,
             send_sems, recv_sems):
        me = lax.axis_index("i")

        bsem = pltpu.get_barrier_semaphore()
        for d in range(1, N_DEV):
            pl.semaphore_signal(
                bsem, inc=1,
                device_id=((me + d) % N_DEV,),
                device_id_type=pl.DeviceIdType.MESH,
            )
        pl.semaphore_wait(bsem, N_DEV - 1)

        win_refs = [win0_ref, win1_ref, win2_ref]
        wout_refs = [wout0_ref, wout1_ref, wout2_ref]
        sends = []

        def rs_send(L, sub, dd, blk_bf):
            tgt = (me + dd) % N_DEV
            p_send[L, sub, dd] = blk_bf
            rdma = pltpu.make_async_remote_copy(
                src_ref=p_send.at[L, sub, dd],
                dst_ref=rs_buf.at[L, sub, N_DEV - dd],
                send_sem=send_sems.at[2 * L, sub, dd],
                recv_sem=recv_sems.at[2 * L, sub, N_DEV - dd],
                device_id=(tgt,),
                device_id_type=pl.DeviceIdType.MESH,
            )
            rdma.start()
            sends.append(rdma)

        def wait_recv(phase, sub, dd, buf_ref):
            recv = pltpu.make_async_remote_copy(
                src_ref=buf_ref, dst_ref=buf_ref,
                send_sem=send_sems.at[phase, sub, dd],
                recv_sem=recv_sems.at[phase, sub, dd],
                device_id=(me,),
                device_id_type=pl.DeviceIdType.MESH,
            )
            recv.wait_recv()

        w_in0 = win_refs[0][...]
        for sub in range(N_SUB):
            for dd in (2, 1, 3):
                s = (me + dd) % N_DEV
                blk = jnp.dot(
                    x_ref[pl.ds(s * Bq + sub * Bs, Bs), :],
                    w_in0, preferred_element_type=jnp.float32,
                ).astype(_BF)
                rs_send(0, sub, dd, blk)
        acc = [
            jnp.dot(
                x_ref[pl.ds(me * Bq + sub * Bs, Bs), :],
                w_in0, preferred_element_type=jnp.float32,
            )
            for sub in range(N_SUB)
        ]

        for L in range(3):
            p_ag = 2 * L + 1
            w_out = wout_refs[L][...]
            w_in_next = win_refs[L + 1][...] if L < 2 else None
            acc_next = [None] * N_SUB

            for sub in range(N_SUB):
                a = acc[sub]
                for dd in range(1, N_DEV):
                    wait_recv(2 * L, sub, dd, rs_buf.at[L, sub, dd])
                    a = a + rs_buf[L, sub, dd].astype(jnp.float32)
                hown_buf[L, sub] = jnp.maximum(a, 0.0).astype(_BF)

                for dd in (2, 1, 3):
                    tgt = (me + dd) % N_DEV
                    rdma = pltpu.make_async_remote_copy(
                        src_ref=hown_buf.at[L, sub],
                        dst_ref=hg_buf.at[L, sub, N_DEV - dd],
                        send_sem=send_sems.at[p_ag, sub, dd],
                        recv_sem=recv_sems.at[p_ag, sub, N_DEV - dd],
                        device_id=(tgt,),
                        device_id_type=pl.DeviceIdType.MESH,
                    )
                    rdma.start()
                    sends.append(rdma)

                xnext_own = jnp.dot(
                    hown_buf[L, sub], w_out,
                    preferred_element_type=jnp.float32,
                )
                if L < 2:
                    acc_next[sub] = jnp.dot(
                        xnext_own.astype(_BF), w_in_next,
                        preferred_element_type=jnp.float32,
                    )
                else:
                    out_ref[pl.ds(me * Bq + sub * Bs, Bs), :] = xnext_own

            for sub in range(N_SUB):
                for dd in range(1, N_DEV):
                    wait_recv(p_ag, sub, dd, hg_buf.at[L, sub, dd])
                    s = (me + dd) % N_DEV
                    xnext_s = jnp.dot(
                        hg_buf[L, sub, dd], w_out,
                        preferred_element_type=jnp.float32,
                    )
                    if L < 2:
                        blk = jnp.dot(
                            xnext_s.astype(_BF), w_in_next,
                            preferred_element_type=jnp.float32,
                        ).astype(_BF)
                        rs_send(L + 1, sub, dd, blk)
                    else:
                        out_ref[pl.ds(s * Bq + sub * Bs, Bs), :] = xnext_s

            if L < 2:
                acc = acc_next

        for rdma in sends:
            rdma.wait_send()

    args_bf = [a.astype(_BF) for a in (x, Win0, Wout0, Win1, Wout1, Win2, Wout2)]
    return pl.pallas_call(
        body,
        out_shape=jax.ShapeDtypeStruct((B, d_sh), jnp.float32),
        in_specs=[pl.BlockSpec(memory_space=pltpu.VMEM)] * 7,
        out_specs=pl.BlockSpec(memory_space=pltpu.VMEM),
        scratch_shapes=[
            pltpu.VMEM((3, N_SUB, N_DEV, Bs, H), _BF),
            pltpu.VMEM((3, N_SUB, N_DEV, Bs, H), _BF),
            pltpu.VMEM((3, N_SUB, Bs, H), _BF),
            pltpu.VMEM((3, N_SUB, N_DEV, Bs, H), _BF),
            pltpu.SemaphoreType.DMA((6, N_SUB, N_DEV)),
            pltpu.SemaphoreType.DMA((6, N_SUB, N_DEV)),
        ],
        compiler_params=pltpu.CompilerParams(collective_id=0),
    )(*args_bf)
